# Initial kernel scaffold; baseline (speedup 1.0000x reference)
import jax
import jax.numpy as jnp
from jax.experimental import pallas as pl


def kernel(x, edge_index, edge_attr, batch, W1, We1, a_src1, a_dst1, a_e1, b1, W2, We2, a_src2, a_dst2, a_e2, b2):
    raise NotImplementedError("write your pallas kernel here")


# speedup vs baseline: 9.9681x; 9.9681x over previous
import functools

import jax
import jax.numpy as jnp
from jax import lax
from jax.experimental import pallas as pl
from jax.experimental.pallas import tpu as pltpu
from jax.experimental.pallas import tpu_sc as plsc

N = 10000
E = 320000
FIN = 128
C = 64
NG = 64

NC = 2
NS = 16
NW = NC * NS
EPW = E // NW
K = 80
NCH = EPW // K
RPS = N // NS

BN = 1000
BE = 6400


def _prep_body(x_ref, w_ref, a2_ref, xl_ref, aa_ref):
    xl = jnp.dot(x_ref[...], w_ref[...], preferred_element_type=jnp.float32)
    xl_ref[...] = xl
    aa_ref[...] = jnp.dot(xl, a2_ref[...], preferred_element_type=jnp.float32)


def _prep(x, W, A2):
    fin = x.shape[1]
    return pl.pallas_call(
        _prep_body,
        grid=(N // BN,),
        in_specs=[
            pl.BlockSpec((BN, fin), lambda i: (i, 0)),
            pl.BlockSpec((fin, C), lambda i: (0, 0)),
            pl.BlockSpec((C, 8), lambda i: (0, 0)),
        ],
        out_specs=[
            pl.BlockSpec((BN, C), lambda i: (i, 0)),
            pl.BlockSpec((BN, 8), lambda i: (i, 0)),
        ],
        out_shape=[
            jax.ShapeDtypeStruct((N, C), jnp.float32),
            jax.ShapeDtypeStruct((N, 8), jnp.float32),
        ],
    )(x, W, A2)


def _edge_body(eat_ref, we_ref, ae_ref, g_ref):
    blk = eat_ref[...]
    rows = []
    for l in range(2):
        w = jnp.sum(we_ref[l] * ae_ref[:, l][None, :], axis=1)
        g = w[0] * blk[0:1, :]
        for kk in range(1, 4):
            g = g + w[kk] * blk[kk:kk + 1, :]
        rows.append(g)
    z = jnp.zeros_like(blk[0:6, :])
    g_ref[...] = jnp.concatenate(rows + [z], axis=0)


def _edge_scalars(eat8, WE, AE):
    return pl.pallas_call(
        _edge_body,
        grid=(E // BE,),
        in_specs=[
            pl.BlockSpec((8, BE), lambda i: (0, i)),
            pl.BlockSpec((2, 4, C), lambda i: (0, 0, 0)),
            pl.BlockSpec((C, 2), lambda i: (0, 0)),
        ],
        out_specs=pl.BlockSpec((8, BE), lambda i: (0, i)),
        out_shape=jax.ShapeDtypeStruct((8, E), jnp.float32),
    )(eat8, WE, AE)


MC = 25


def _sc_body(sdg_h, aa_h, xl_h, enum_o, scal_o,
             sdg_v, aa_v, acc_v, rows_v, enum_sh, sem):
    c = lax.axis_index("c")
    s = lax.axis_index("s")
    w = s * NC + c

    zero16 = jnp.zeros((16,), jnp.float32)
    ones16 = jnp.ones((16,), jnp.float32)
    lane = lax.iota(jnp.int32, 16)

    for r in range(K):
        for q in range(4):
            rows_v[r, pl.ds(q * 16, 16)] = zero16

    @pl.loop(0, (4 * N) // 16)
    def _zacc(i):
        acc_v[pl.ds(i * 16, 16)] = zero16

    pltpu.sync_copy(aa_h, aa_v)

    @pl.loop(0, (N // K + NS - 1) // NS)
    def _zsh(i):
        ch = s + i * NS

        @pl.when(ch < N // K)
        def _():
            pltpu.sync_copy(rows_v, enum_sh.at[pl.ds(ch * K, K)])
    plsc.subcore_barrier()

    @pl.loop(0, NCH // MC)
    def _mc(m):
        pltpu.sync_copy(sdg_h.at[w, pl.ds(m * MC, MC)], sdg_v)

        @pl.loop(0, MC)
        def _chunk(t):
            pltpu.async_copy(xl_h.at[sdg_v.at[t, 0]], rows_v, sem).wait()
            for j in range(K // 16):
                sl = pl.ds(j * 16, 16)
                sv = sdg_v[t, 0, sl]
                dv = sdg_v[t, 1, sl]
                gv = plsc.bitcast(sdg_v[t, 2, sl], jnp.float32)
                g2v = plsc.bitcast(sdg_v[t, 3, sl], jnp.float32)
                av = (plsc.load_gather(aa_v, [sv * 2])
                      + plsc.load_gather(aa_v, [dv * 2 + 1]) + gv)
                av = jnp.where(av > 0, av, 0.2 * av)
                ex = jnp.exp(av)
                d4 = dv * 4
                plsc.addupdate_scatter(acc_v, [d4], ex)
                plsc.addupdate_scatter(acc_v, [d4 + 1], ones16)
                plsc.addupdate_scatter(acc_v, [d4 + 2], gv)
                plsc.addupdate_scatter(acc_v, [d4 + 3], g2v)
                rowi = lane + j * 16
                for cc in range(C):
                    coli = jnp.full((16,), cc, jnp.int32)
                    col = plsc.load_gather(rows_v, [rowi, coli])
                    plsc.store_scatter(rows_v, [rowi, coli], col * ex)
            pltpu.sync_copy(rows_v, enum_sh.at[sdg_v.at[t, 1]], add=True)

    plsc.subcore_barrier()
    pltpu.sync_copy(acc_v, scal_o.at[w])

    @pl.loop(0, (N // K + NS - 1) // NS)
    def _dr(i):
        ch = s + i * NS

        @pl.when(ch < N // K)
        def _():
            pltpu.sync_copy(enum_sh.at[pl.ds(ch * K, K)],
                            enum_o.at[c, pl.ds(ch * K, K)])


def _sc_pass(sdg, aaflat, xl):
    mesh = plsc.VectorSubcoreMesh(core_axis_name="c", subcore_axis_name="s",
                                  num_cores=NC, num_subcores=NS)
    fn = functools.partial(
        pl.kernel,
        out_type=[
            jax.ShapeDtypeStruct((NC, N, C), jnp.float32),
            jax.ShapeDtypeStruct((NW, 4 * N), jnp.float32),
        ],
        mesh=mesh,
        compiler_params=pltpu.CompilerParams(needs_layout_passes=False,
                                             use_tc_tiling_on_sc=False),
        scratch_types=[
            pltpu.VMEM((MC, 4, K), jnp.int32),
            pltpu.VMEM((2 * N,), jnp.float32),
            pltpu.VMEM((4 * N,), jnp.float32),
            pltpu.VMEM((K, C), jnp.float32),
            pltpu.VMEM_SHARED((N, C), jnp.float32),
            pltpu.SemaphoreType.DMA,
        ],
    )(_sc_body)
    return fn(sdg, aaflat, xl)


def _post1_body(xl1_ref, aa_ref, en_ref, sc_ref, w2_ref, a2_ref, b1_ref,
                xl2_ref, aa2_ref):
    red = jnp.sum(sc_ref[...], axis=0)
    cnt = jnp.maximum(red[:, 1:2], 1.0)
    s1 = red[:, 2:3] / cnt
    s2 = red[:, 3:4] / cnt
    al = aa_ref[:, 0:1] + aa_ref[:, 1:2] + s1
    al = jnp.where(al > 0, al, 0.2 * al)
    exs = jnp.exp(al)
    rden = 1.0 / (exs + red[:, 0:1])
    ensum = en_ref[0] + en_ref[1]
    h = (ensum + exs * xl1_ref[...]) * rden + b1_ref[0:1, :]
    xl2 = jnp.dot(h, w2_ref[...], preferred_element_type=jnp.float32)
    xl2_ref[...] = xl2
    aa2 = jnp.dot(xl2, a2_ref[...], preferred_element_type=jnp.float32)
    z = jnp.zeros_like(aa2[:, 0:5])
    aa2_ref[...] = jnp.concatenate([aa2[:, 0:2], s2, z], axis=1)


def _post1(xl1, aa1, enum1, scal1, W2, A22, b1row):
    return pl.pallas_call(
        _post1_body,
        grid=(N // BN,),
        in_specs=[
            pl.BlockSpec((BN, C), lambda i: (i, 0)),
            pl.BlockSpec((BN, 8), lambda i: (i, 0)),
            pl.BlockSpec((NC, BN, C), lambda i: (0, i, 0)),
            pl.BlockSpec((NW, BN, 4), lambda i: (0, i, 0)),
            pl.BlockSpec((C, C), lambda i: (0, 0)),
            pl.BlockSpec((C, 8), lambda i: (0, 0)),
            pl.BlockSpec((8, C), lambda i: (0, 0)),
        ],
        out_specs=[
            pl.BlockSpec((BN, C), lambda i: (i, 0)),
            pl.BlockSpec((BN, 8), lambda i: (i, 0)),
        ],
        out_shape=[
            jax.ShapeDtypeStruct((N, C), jnp.float32),
            jax.ShapeDtypeStruct((N, 8), jnp.float32),
        ],
    )(xl1, aa1, enum1, scal1, W2, A22, b1row)


def _post2_body(xl2_ref, aa_ref, en_ref, sc_ref, b2_ref, bat_ref, out_ref):
    red = jnp.sum(sc_ref[...], axis=0)
    al = aa_ref[:, 0:1] + aa_ref[:, 1:2] + aa_ref[:, 2:3]
    al = jnp.where(al > 0, al, 0.2 * al)
    exs = jnp.exp(al)
    rden = 1.0 / (exs + red[:, 0:1])
    ensum = en_ref[0] + en_ref[1]
    h = (ensum + exs * xl2_ref[...]) * rden + b2_ref[0:1, :]
    bat = bat_ref[0]
    gid = lax.broadcasted_iota(jnp.int32, (NG, BN), 0)
    oh = jnp.where(gid == bat, 1.0, 0.0)
    acc = jnp.dot(oh, h, preferred_element_type=jnp.float32)

    @pl.when(pl.program_id(0) == 0)
    def _():
        out_ref[...] = jnp.zeros_like(out_ref)
    out_ref[...] += acc


def _post2(xl2, aa2, enum2, scal2, b2row, bat3):
    return pl.pallas_call(
        _post2_body,
        grid=(N // BN,),
        in_specs=[
            pl.BlockSpec((BN, C), lambda i: (i, 0)),
            pl.BlockSpec((BN, 8), lambda i: (i, 0)),
            pl.BlockSpec((NC, BN, C), lambda i: (0, i, 0)),
            pl.BlockSpec((NW, BN, 4), lambda i: (0, i, 0)),
            pl.BlockSpec((8, C), lambda i: (0, 0)),
            pl.BlockSpec((1, 1, BN), lambda i: (i, 0, 0)),
        ],
        out_specs=pl.BlockSpec((NG, NG), lambda i: (0, 0)),
        out_shape=jax.ShapeDtypeStruct((NG, NG), jnp.float32),
    )(xl2, aa2, enum2, scal2, b2row, bat3)


def kernel(x, edge_index, edge_attr, batch, W1, We1, a_src1, a_dst1, a_e1, b1,
           W2, We2, a_src2, a_dst2, a_e2, b2):
    x = x.astype(jnp.float32)
    edge_attr = edge_attr.astype(jnp.float32)
    src3 = edge_index[0].reshape(NW, NCH, K)
    dst3 = edge_index[1].reshape(NW, NCH, K)

    zc = jnp.zeros((C,), jnp.float32)
    A21 = jnp.stack([a_src1, a_dst1, zc, zc, zc, zc, zc, zc], axis=1)
    A22 = jnp.stack([a_src2, a_dst2, zc, zc, zc, zc, zc, zc], axis=1)
    WE = jnp.stack([We1, We2], axis=0)
    AE = jnp.stack([a_e1, a_e2], axis=1)
    eat8 = jnp.concatenate(
        [edge_attr.T, jnp.zeros((4, E), jnp.float32)], axis=0)
    b1row = jnp.broadcast_to(b1[None, :], (8, C))
    b2row = jnp.broadcast_to(b2[None, :], (8, C))
    bat3 = batch.reshape(N // BN, 1, BN)

    g8 = _edge_scalars(eat8, WE, AE)
    ga3 = lax.bitcast_convert_type(g8[0], jnp.int32).reshape(NW, NCH, K)
    gb3 = lax.bitcast_convert_type(g8[1], jnp.int32).reshape(NW, NCH, K)
    sdg1 = jnp.stack([src3, dst3, ga3, gb3], axis=2)
    sdg2 = jnp.stack([src3, dst3, gb3, gb3], axis=2)

    xl1, aa1 = _prep(x, W1, A21)
    aaflat1 = aa1[:, 0:2].reshape(2 * N)
    enum1, scal1 = _sc_pass(sdg1, aaflat1, xl1)

    xl2, aa2 = _post1(xl1, aa1, enum1, scal1.reshape(NW, N, 4), W2, A22, b1row)
    aaflat2 = aa2[:, 0:2].reshape(2 * N)
    enum2, scal2 = _sc_pass(sdg2, aaflat2, xl2)

    return _post2(xl2, aa2, enum2, scal2.reshape(NW, N, 4), b2row, bat3)


# speedup vs baseline: 10.2448x; 1.0278x over previous
import functools

import jax
import jax.numpy as jnp
from jax import lax
from jax.experimental import pallas as pl
from jax.experimental.pallas import tpu as pltpu
from jax.experimental.pallas import tpu_sc as plsc

N = 10000
E = 320000
FIN = 128
C = 64
NG = 64

NC = 2
NS = 16
NW = NC * NS
EPW = E // NW
K = 80
NCH = EPW // K
RPS = N // NS

BN = 1000
BE = 6400


def _prep_body(x_ref, w_ref, a2_ref, xl_ref, aa_ref):
    xl = jnp.dot(x_ref[...], w_ref[...], preferred_element_type=jnp.float32)
    xl_ref[...] = xl
    aa_ref[...] = jnp.dot(xl, a2_ref[...], preferred_element_type=jnp.float32)


def _prep(x, W, A2):
    fin = x.shape[1]
    return pl.pallas_call(
        _prep_body,
        grid=(N // BN,),
        in_specs=[
            pl.BlockSpec((BN, fin), lambda i: (i, 0)),
            pl.BlockSpec((fin, C), lambda i: (0, 0)),
            pl.BlockSpec((C, 8), lambda i: (0, 0)),
        ],
        out_specs=[
            pl.BlockSpec((BN, C), lambda i: (i, 0)),
            pl.BlockSpec((BN, 8), lambda i: (i, 0)),
        ],
        out_shape=[
            jax.ShapeDtypeStruct((N, C), jnp.float32),
            jax.ShapeDtypeStruct((N, 8), jnp.float32),
        ],
    )(x, W, A2)


def _edge_body(eat_ref, we_ref, ae_ref, g_ref):
    blk = eat_ref[...]
    rows = []
    for l in range(2):
        w = jnp.sum(we_ref[l] * ae_ref[:, l][None, :], axis=1)
        g = w[0] * blk[0:1, :]
        for kk in range(1, 4):
            g = g + w[kk] * blk[kk:kk + 1, :]
        rows.append(g)
    z = jnp.zeros_like(blk[0:6, :])
    g_ref[...] = jnp.concatenate(rows + [z], axis=0)


def _edge_scalars(eat8, WE, AE):
    return pl.pallas_call(
        _edge_body,
        grid=(E // BE,),
        in_specs=[
            pl.BlockSpec((8, BE), lambda i: (0, i)),
            pl.BlockSpec((2, 4, C), lambda i: (0, 0, 0)),
            pl.BlockSpec((C, 2), lambda i: (0, 0)),
        ],
        out_specs=pl.BlockSpec((8, BE), lambda i: (0, i)),
        out_shape=jax.ShapeDtypeStruct((8, E), jnp.float32),
    )(eat8, WE, AE)


MC = 25


def _sc_body(sdg_h, aa_h, xl_h, enum_o, scal_o,
             sdg_v, aa_v, acc_v, rows_v, enum_sh, sem_g, sem_s, sem_m):
    c = lax.axis_index("c")
    s = lax.axis_index("s")
    w = s * NC + c

    zero16 = jnp.zeros((16,), jnp.float32)
    ones16 = jnp.ones((16,), jnp.float32)
    lane = lax.iota(jnp.int32, 16)

    for bb in range(2):
        for r in range(K):
            for q in range(4):
                rows_v[bb, r, pl.ds(q * 16, 16)] = zero16

    @pl.loop(0, (4 * N) // 16)
    def _zacc(i):
        acc_v[pl.ds(i * 16, 16)] = zero16

    pltpu.sync_copy(aa_h, aa_v)

    @pl.loop(0, (N // K + NS - 1) // NS)
    def _zsh(i):
        ch = s + i * NS

        @pl.when(ch < N // K)
        def _():
            pltpu.sync_copy(rows_v.at[0], enum_sh.at[pl.ds(ch * K, K)])
    plsc.subcore_barrier()

    def _gather(mb, tt, rb):
        pltpu.async_copy(xl_h.at[sdg_v.at[mb, tt, 0]], rows_v.at[rb], sem_g)

    def _wait_gather(mb, tt, rb):
        pltpu.make_async_copy(xl_h.at[sdg_v.at[mb, tt, 0]], rows_v.at[rb],
                              sem_g).wait()

    def _scatter(mb, tt, rb):
        pltpu.async_copy(rows_v.at[rb], enum_sh.at[sdg_v.at[mb, tt, 1]],
                         sem_s, add=True)

    def _wait_scatter(mb, tt, rb):
        pltpu.make_async_copy(rows_v.at[rb], enum_sh.at[sdg_v.at[mb, tt, 1]],
                              sem_s).wait()

    pltpu.sync_copy(sdg_h.at[w, pl.ds(0, MC)], sdg_v.at[0])
    _gather(0, 0, 0)

    @pl.loop(0, NCH // MC)
    def _mc(m):
        bm = lax.rem(m, 2)

        @pl.when(m > 0)
        def _():
            _wait_scatter(1 - bm, MC - 1, lax.rem(m * MC - 1, 2))

        @pl.when(m < NCH // MC - 1)
        def _():
            pltpu.async_copy(sdg_h.at[w, pl.ds((m + 1) * MC, MC)],
                             sdg_v.at[1 - bm], sem_m)

        @pl.loop(0, MC)
        def _chunk(t):
            gc = m * MC + t
            b = lax.rem(gc, 2)
            _wait_gather(bm, t, b)
            rows_b = rows_v.at[b]
            for j in range(K // 16):
                sl = pl.ds(j * 16, 16)
                sv = sdg_v[bm, t, 0, sl]
                dv = sdg_v[bm, t, 1, sl]
                gv = plsc.bitcast(sdg_v[bm, t, 2, sl], jnp.float32)
                g2v = plsc.bitcast(sdg_v[bm, t, 3, sl], jnp.float32)
                av = (plsc.load_gather(aa_v, [sv * 2])
                      + plsc.load_gather(aa_v, [dv * 2 + 1]) + gv)
                av = jnp.where(av > 0, av, 0.2 * av)
                ex = jnp.exp(av)
                d4 = dv * 4
                plsc.addupdate_scatter(acc_v, [d4], ex)
                plsc.addupdate_scatter(acc_v, [d4 + 1], ones16)
                plsc.addupdate_scatter(acc_v, [d4 + 2], gv)
                plsc.addupdate_scatter(acc_v, [d4 + 3], g2v)
                rowi = lane + j * 16
                for cc in range(C):
                    coli = jnp.full((16,), cc, jnp.int32)
                    col = plsc.load_gather(rows_b, [rowi, coli])
                    plsc.store_scatter(rows_b, [rowi, coli], col * ex)
            _scatter(bm, t, b)

            @pl.when(t > 0)
            def _():
                _wait_scatter(bm, t - 1, 1 - b)

            @pl.when(jnp.logical_and(t == MC - 1, m < NCH // MC - 1))
            def _():
                pltpu.make_async_copy(sdg_h.at[w, pl.ds((m + 1) * MC, MC)],
                                      sdg_v.at[1 - bm], sem_m).wait()
                _gather(1 - bm, 0, 1 - b)

            @pl.when(jnp.logical_and(t < MC - 1, gc < NCH - 1))
            def _():
                _gather(bm, t + 1, 1 - b)

        return None

    _wait_scatter((NCH // MC - 1) % 2, MC - 1, (NCH - 1) % 2)

    plsc.subcore_barrier()
    pltpu.sync_copy(acc_v, scal_o.at[w])

    @pl.loop(0, (N // K + NS - 1) // NS)
    def _dr(i):
        ch = s + i * NS

        @pl.when(ch < N // K)
        def _():
            pltpu.sync_copy(enum_sh.at[pl.ds(ch * K, K)],
                            enum_o.at[c, pl.ds(ch * K, K)])


def _sc_pass(sdg, aaflat, xl):
    mesh = plsc.VectorSubcoreMesh(core_axis_name="c", subcore_axis_name="s",
                                  num_cores=NC, num_subcores=NS)
    fn = functools.partial(
        pl.kernel,
        out_type=[
            jax.ShapeDtypeStruct((NC, N, C), jnp.float32),
            jax.ShapeDtypeStruct((NW, 4 * N), jnp.float32),
        ],
        mesh=mesh,
        compiler_params=pltpu.CompilerParams(needs_layout_passes=False,
                                             use_tc_tiling_on_sc=False),
        scratch_types=[
            pltpu.VMEM((2, MC, 4, K), jnp.int32),
            pltpu.VMEM((2 * N,), jnp.float32),
            pltpu.VMEM((4 * N,), jnp.float32),
            pltpu.VMEM((2, K, C), jnp.float32),
            pltpu.VMEM_SHARED((N, C), jnp.float32),
            pltpu.SemaphoreType.DMA,
            pltpu.SemaphoreType.DMA,
            pltpu.SemaphoreType.DMA,
        ],
    )(_sc_body)
    return fn(sdg, aaflat, xl)


def _post1_body(xl1_ref, aa_ref, en_ref, sc_ref, w2_ref, a2_ref, b1_ref,
                xl2_ref, aa2_ref):
    red = jnp.sum(sc_ref[...], axis=0)
    cnt = jnp.maximum(red[:, 1:2], 1.0)
    s1 = red[:, 2:3] / cnt
    s2 = red[:, 3:4] / cnt
    al = aa_ref[:, 0:1] + aa_ref[:, 1:2] + s1
    al = jnp.where(al > 0, al, 0.2 * al)
    exs = jnp.exp(al)
    rden = 1.0 / (exs + red[:, 0:1])
    ensum = en_ref[0] + en_ref[1]
    h = (ensum + exs * xl1_ref[...]) * rden + b1_ref[0:1, :]
    xl2 = jnp.dot(h, w2_ref[...], preferred_element_type=jnp.float32)
    xl2_ref[...] = xl2
    aa2 = jnp.dot(xl2, a2_ref[...], preferred_element_type=jnp.float32)
    z = jnp.zeros_like(aa2[:, 0:5])
    aa2_ref[...] = jnp.concatenate([aa2[:, 0:2], s2, z], axis=1)


def _post1(xl1, aa1, enum1, scal1, W2, A22, b1row):
    return pl.pallas_call(
        _post1_body,
        grid=(N // BN,),
        in_specs=[
            pl.BlockSpec((BN, C), lambda i: (i, 0)),
            pl.BlockSpec((BN, 8), lambda i: (i, 0)),
            pl.BlockSpec((NC, BN, C), lambda i: (0, i, 0)),
            pl.BlockSpec((NW, BN, 4), lambda i: (0, i, 0)),
            pl.BlockSpec((C, C), lambda i: (0, 0)),
            pl.BlockSpec((C, 8), lambda i: (0, 0)),
            pl.BlockSpec((8, C), lambda i: (0, 0)),
        ],
        out_specs=[
            pl.BlockSpec((BN, C), lambda i: (i, 0)),
            pl.BlockSpec((BN, 8), lambda i: (i, 0)),
        ],
        out_shape=[
            jax.ShapeDtypeStruct((N, C), jnp.float32),
            jax.ShapeDtypeStruct((N, 8), jnp.float32),
        ],
    )(xl1, aa1, enum1, scal1, W2, A22, b1row)


def _post2_body(xl2_ref, aa_ref, en_ref, sc_ref, b2_ref, bat_ref, out_ref):
    red = jnp.sum(sc_ref[...], axis=0)
    al = aa_ref[:, 0:1] + aa_ref[:, 1:2] + aa_ref[:, 2:3]
    al = jnp.where(al > 0, al, 0.2 * al)
    exs = jnp.exp(al)
    rden = 1.0 / (exs + red[:, 0:1])
    ensum = en_ref[0] + en_ref[1]
    h = (ensum + exs * xl2_ref[...]) * rden + b2_ref[0:1, :]
    bat = bat_ref[0]
    gid = lax.broadcasted_iota(jnp.int32, (NG, BN), 0)
    oh = jnp.where(gid == bat, 1.0, 0.0)
    acc = jnp.dot(oh, h, preferred_element_type=jnp.float32)

    @pl.when(pl.program_id(0) == 0)
    def _():
        out_ref[...] = jnp.zeros_like(out_ref)
    out_ref[...] += acc


def _post2(xl2, aa2, enum2, scal2, b2row, bat3):
    return pl.pallas_call(
        _post2_body,
        grid=(N // BN,),
        in_specs=[
            pl.BlockSpec((BN, C), lambda i: (i, 0)),
            pl.BlockSpec((BN, 8), lambda i: (i, 0)),
            pl.BlockSpec((NC, BN, C), lambda i: (0, i, 0)),
            pl.BlockSpec((NW, BN, 4), lambda i: (0, i, 0)),
            pl.BlockSpec((8, C), lambda i: (0, 0)),
            pl.BlockSpec((1, 1, BN), lambda i: (i, 0, 0)),
        ],
        out_specs=pl.BlockSpec((NG, NG), lambda i: (0, 0)),
        out_shape=jax.ShapeDtypeStruct((NG, NG), jnp.float32),
    )(xl2, aa2, enum2, scal2, b2row, bat3)


def kernel(x, edge_index, edge_attr, batch, W1, We1, a_src1, a_dst1, a_e1, b1,
           W2, We2, a_src2, a_dst2, a_e2, b2):
    x = x.astype(jnp.float32)
    edge_attr = edge_attr.astype(jnp.float32)
    src3 = edge_index[0].reshape(NW, NCH, K)
    dst3 = edge_index[1].reshape(NW, NCH, K)

    zc = jnp.zeros((C,), jnp.float32)
    A21 = jnp.stack([a_src1, a_dst1, zc, zc, zc, zc, zc, zc], axis=1)
    A22 = jnp.stack([a_src2, a_dst2, zc, zc, zc, zc, zc, zc], axis=1)
    WE = jnp.stack([We1, We2], axis=0)
    AE = jnp.stack([a_e1, a_e2], axis=1)
    eat8 = jnp.concatenate(
        [edge_attr.T, jnp.zeros((4, E), jnp.float32)], axis=0)
    b1row = jnp.broadcast_to(b1[None, :], (8, C))
    b2row = jnp.broadcast_to(b2[None, :], (8, C))
    bat3 = batch.reshape(N // BN, 1, BN)

    g8 = _edge_scalars(eat8, WE, AE)
    ga3 = lax.bitcast_convert_type(g8[0], jnp.int32).reshape(NW, NCH, K)
    gb3 = lax.bitcast_convert_type(g8[1], jnp.int32).reshape(NW, NCH, K)
    sdg1 = jnp.stack([src3, dst3, ga3, gb3], axis=2)
    sdg2 = jnp.stack([src3, dst3, gb3, gb3], axis=2)

    xl1, aa1 = _prep(x, W1, A21)
    aaflat1 = aa1[:, 0:2].reshape(2 * N)
    enum1, scal1 = _sc_pass(sdg1, aaflat1, xl1)

    xl2, aa2 = _post1(xl1, aa1, enum1, scal1.reshape(NW, N, 4), W2, A22, b1row)
    aaflat2 = aa2[:, 0:2].reshape(2 * N)
    enum2, scal2 = _sc_pass(sdg2, aaflat2, xl2)

    return _post2(xl2, aa2, enum2, scal2.reshape(NW, N, 4), b2row, bat3)
